# baseline (speedup 1.0000x reference)
import jax
import jax.numpy as jnp
from jax import lax
from jax.experimental import pallas as pl
from jax.experimental.pallas import tpu as pltpu
from jax.experimental.pallas import tpu_sc as plsc

_B = 16384
_D = 32
_NW = 32
_BPW = _B // _NW
_MU = 10000000.0 / (10000000.0 + 1000000.0 * 4.0)


def _mf_body(uid_hbm, iid_hbm, ue_hbm, ie_hbm, bu_hbm, bi_hbm, out_hbm,
             uid_v, iid_v, u_rows, i_rows, bu_v, bi_v, out_v, sem):
  wid = lax.axis_index("s") * 2 + lax.axis_index("c")
  base = wid * _BPW

  pltpu.sync_copy(uid_hbm.at[pl.ds(base, _BPW)], uid_v)
  pltpu.sync_copy(iid_hbm.at[pl.ds(base, _BPW)], iid_v)

  for j in range(_BPW // 128):
    sl = pl.ds(j * 128, 128)
    pltpu.async_copy(bu_hbm.at[uid_v.at[sl]], bu_v.at[sl], sem)
    pltpu.async_copy(bi_hbm.at[iid_v.at[sl]], bi_v.at[sl], sem)
  pltpu.make_async_copy(bu_hbm.at[pl.ds(0, _BPW)], bu_v, sem).wait()
  pltpu.make_async_copy(bi_hbm.at[pl.ds(0, _BPW)], bi_v, sem).wait()

  lane = lax.iota(jnp.int32, 16)
  bitrev = (((lane & 1) << 3) | ((lane & 2) << 1) |
            ((lane & 4) >> 1) | ((lane & 8) >> 3))

  def _perm(v, idx):
    return lax.gather(
        v, idx[:, None],
        lax.GatherDimensionNumbers(offset_dims=(), collapsed_slice_dims=(0,),
                                   start_index_map=(0,)),
        slice_sizes=(1,), mode=lax.GatherScatterMode.PROMISE_IN_BOUNDS)

  def _shuf(v, k):
    return _perm(v, lane ^ k)

  pp = _BPW // 2
  for p in range(2):
    pbase = p * pp

    def fire(g, carry, pbase=pbase):
      off = g * 16
      uvec = uid_v[pl.ds(pbase + off, 16)]
      ivec = iid_v[pl.ds(pbase + off, 16)]
      for r in range(16):
        pltpu.async_copy(ue_hbm.at[pl.ds(uvec[r], 1)],
                         u_rows.at[pl.ds(off + r, 1)], sem)
        pltpu.async_copy(ie_hbm.at[pl.ds(ivec[r], 1)],
                         i_rows.at[pl.ds(off + r, 1)], sem)
      return carry
    lax.fori_loop(0, pp // 16, fire, 0)
    pltpu.make_async_copy(ue_hbm.at[pl.ds(0, pp)], u_rows, sem).wait()
    pltpu.make_async_copy(ie_hbm.at[pl.ds(0, pp)], i_rows, sem).wait()

    def g_body(g, carry, pbase=pbase):
      off = g * 16
      vecs = []
      for r in range(16):
        row = off + r
        vecs.append(u_rows[row, pl.ds(0, 16)] * i_rows[row, pl.ds(0, 16)] +
                    u_rows[row, pl.ds(16, 16)] * i_rows[row, pl.ds(16, 16)])
      for k in (8, 4, 2, 1):
        m = (lane & k) == 0
        vecs = [jnp.where(m, x + _shuf(x, k), y + _shuf(y, k))
                for x, y in zip(vecs[0::2], vecs[1::2])]
      dots = _perm(vecs[0], bitrev)
      out_v[pl.ds(pbase + off, 16)] = (dots + bu_v[pl.ds(pbase + off, 16)] +
                                       bi_v[pl.ds(pbase + off, 16)] + _MU)
      return carry
    lax.fori_loop(0, pp // 16, g_body, 0)

  pltpu.sync_copy(out_v, out_hbm.at[pl.ds(base, _BPW)])


_TN = 1000000
_TBLK = 2048


def _tp_body(x_ref, o_ref):
  o_ref[...] = x_ref[...].T


def _transpose_tc(t):
  return pl.pallas_call(
      _tp_body,
      grid=(pl.cdiv(_TN, _TBLK),),
      in_specs=[pl.BlockSpec((_D, _TBLK), lambda i: (0, i))],
      out_specs=pl.BlockSpec((_TBLK, _D), lambda i: (i, 0)),
      out_shape=jax.ShapeDtypeStruct((_TN, _D), jnp.float32),
  )(t)


def kernel(x, user_embedding, item_embedding, b_u, b_i):
  uid = x[:, 0].astype(jnp.int32)
  iid = x[:, 1].astype(jnp.int32)
  ue_row = _transpose_tc(user_embedding.T)
  ie_row = _transpose_tc(item_embedding.T)
  mesh = plsc.VectorSubcoreMesh(core_axis_name="c", subcore_axis_name="s")
  mf = pl.kernel(
      _mf_body,
      out_type=jax.ShapeDtypeStruct((_B,), jnp.float32),
      mesh=mesh,
      scratch_types=[
          pltpu.VMEM((_BPW,), jnp.int32),
          pltpu.VMEM((_BPW,), jnp.int32),
          pltpu.VMEM((_BPW // 2, _D), jnp.float32),
          pltpu.VMEM((_BPW // 2, _D), jnp.float32),
          pltpu.VMEM((_BPW,), jnp.float32),
          pltpu.VMEM((_BPW,), jnp.float32),
          pltpu.VMEM((_BPW,), jnp.float32),
          pltpu.SemaphoreType.DMA,
      ],
  )
  return mf(uid, iid, ue_row, ie_row, b_u, b_i)


# speedup vs baseline: 1.5002x; 1.5002x over previous
import jax
import jax.numpy as jnp
from jax import lax
from jax.experimental import pallas as pl
from jax.experimental.pallas import tpu as pltpu
from jax.experimental.pallas import tpu_sc as plsc

_B = 16384
_D = 32
_NW = 32
_BPW = _B // _NW
_MU = 10000000.0 / (10000000.0 + 1000000.0 * 4.0)


def _mf_body(uid_hbm, iid_hbm, ue_hbm, ie_hbm, bu_hbm, bi_hbm, out_hbm,
             uid_v, iid_v, u_rows, i_rows, bu_v, bi_v, out_v, sem):
  wid = lax.axis_index("s") * 2 + lax.axis_index("c")
  base = wid * _BPW

  pltpu.sync_copy(uid_hbm.at[pl.ds(base, _BPW)], uid_v)
  pltpu.sync_copy(iid_hbm.at[pl.ds(base, _BPW)], iid_v)

  for j in range(_BPW // 128):
    sl = pl.ds(j * 128, 128)
    pltpu.async_copy(bu_hbm.at[uid_v.at[sl]], bu_v.at[sl], sem)
    pltpu.async_copy(bi_hbm.at[iid_v.at[sl]], bi_v.at[sl], sem)
  pltpu.make_async_copy(bu_hbm.at[pl.ds(0, _BPW)], bu_v, sem).wait()
  pltpu.make_async_copy(bi_hbm.at[pl.ds(0, _BPW)], bi_v, sem).wait()

  lane = lax.iota(jnp.int32, 16)
  bitrev = (((lane & 1) << 3) | ((lane & 2) << 1) |
            ((lane & 4) >> 1) | ((lane & 8) >> 3))

  def _perm(v, idx):
    return lax.gather(
        v, idx[:, None],
        lax.GatherDimensionNumbers(offset_dims=(), collapsed_slice_dims=(0,),
                                   start_index_map=(0,)),
        slice_sizes=(1,), mode=lax.GatherScatterMode.PROMISE_IN_BOUNDS)

  def _shuf(v, k):
    return _perm(v, lane ^ k)

  pp = _BPW // 2
  for p in range(2):
    pbase = p * pp

    def fire(g, carry, pbase=pbase):
      off = g * 16
      uvec = uid_v[pl.ds(pbase + off, 16)]
      ivec = iid_v[pl.ds(pbase + off, 16)]
      for r in range(16):
        pltpu.async_copy(ue_hbm.at[pl.ds(uvec[r], 1)],
                         u_rows.at[pl.ds(off + r, 1)], sem)
        pltpu.async_copy(ie_hbm.at[pl.ds(ivec[r], 1)],
                         i_rows.at[pl.ds(off + r, 1)], sem)
      return carry
    lax.fori_loop(0, pp // 16, fire, 0)
    pltpu.make_async_copy(ue_hbm.at[pl.ds(0, pp)], u_rows, sem).wait()
    pltpu.make_async_copy(ie_hbm.at[pl.ds(0, pp)], i_rows, sem).wait()

    def g_body(g, carry, pbase=pbase):
      off = g * 16
      vecs = []
      for r in range(16):
        row = off + r
        vecs.append(u_rows[row, pl.ds(0, 16)] * i_rows[row, pl.ds(0, 16)] +
                    u_rows[row, pl.ds(16, 16)] * i_rows[row, pl.ds(16, 16)])
      for k in (8, 4, 2, 1):
        m = (lane & k) == 0
        vecs = [jnp.where(m, x + _shuf(x, k), y + _shuf(y, k))
                for x, y in zip(vecs[0::2], vecs[1::2])]
      dots = _perm(vecs[0], bitrev)
      out_v[pl.ds(pbase + off, 16)] = (dots + bu_v[pl.ds(pbase + off, 16)] +
                                       bi_v[pl.ds(pbase + off, 16)] + _MU)
      return carry
    lax.fori_loop(0, pp // 16, g_body, 0)

  pltpu.sync_copy(out_v, out_hbm.at[pl.ds(base, _BPW)])


def kernel(x, user_embedding, item_embedding, b_u, b_i):
  uid = x[:, 0].astype(jnp.int32)
  iid = x[:, 1].astype(jnp.int32)
  mesh = plsc.VectorSubcoreMesh(core_axis_name="c", subcore_axis_name="s")
  mf = pl.kernel(
      _mf_body,
      out_type=jax.ShapeDtypeStruct((_B,), jnp.float32),
      mesh=mesh,
      scratch_types=[
          pltpu.VMEM((_BPW,), jnp.int32),
          pltpu.VMEM((_BPW,), jnp.int32),
          pltpu.VMEM((_BPW // 2, _D), jnp.float32),
          pltpu.VMEM((_BPW // 2, _D), jnp.float32),
          pltpu.VMEM((_BPW,), jnp.float32),
          pltpu.VMEM((_BPW,), jnp.float32),
          pltpu.VMEM((_BPW,), jnp.float32),
          pltpu.SemaphoreType.DMA,
      ],
  )
  return mf(uid, iid, user_embedding, item_embedding, b_u, b_i)
